# baseline (speedup 1.0000x reference)
import jax
import jax.numpy as jnp
from jax.experimental import pallas as pl
from jax.experimental.pallas import tpu as pltpu

NELE = 90
NAZI = 90
G = NELE * NAZI
K = 256 * 6
R = 100
SCALE = 768.0
BE = 6
NBLK = NELE // BE
PA = 96
BROWS = BE * PA
NMIC = 6
NF = 256


def _doa_from_idx(ridx, doaT_ref):
    ele = ridx // NAZI
    azi = ridx % NAZI
    kk = jax.lax.broadcasted_iota(jnp.int32, (NAZI, 1), 0)
    elev = jnp.sum((kk == ele).astype(jnp.float32) * doaT_ref[:, 0:1],
                   axis=0, keepdims=True)
    aziv = jnp.sum((kk == azi).astype(jnp.float32) * doaT_ref[:, 1:2],
                   axis=0, keepdims=True)
    return jnp.concatenate([elev, aziv], axis=0)


def _ele_dot(tm_s, e, xT_s):
    return jax.lax.dot_general(
        tm_s[e], xT_s[...], (((1,), (0,)), ((), ())),
        preferred_element_type=jnp.float32) / SCALE


def _argmax_update(e, first, m, runmax_s, runidx_s):
    gidx = e * NAZI + jax.lax.broadcasted_iota(jnp.int32, (NAZI, 1), 0)
    bmax = jnp.max(m, axis=0, keepdims=True)
    bidx = jnp.min(jnp.where(m == bmax, gidx, jnp.int32(2**31 - 1)),
                   axis=0, keepdims=True)

    @pl.when(first)
    def _():
        runmax_s[...] = jnp.full((1, R), -jnp.inf, jnp.float32)
        runidx_s[...] = jnp.zeros((1, R), jnp.int32)

    better = bmax > runmax_s[...]
    runmax_s[...] = jnp.where(better, bmax, runmax_s[...])
    runidx_s[...] = jnp.where(better, bidx, runidx_s[...])


def _start_block_dmas(tv_hbm, tm_s, b, sem):
    for m in range(NMIC):
        pltpu.make_async_copy(
            tv_hbm.at[pl.ds(b * BE, BE), m, :, :],
            tm_s.at[pl.ds(b * BE, BE), :, pl.ds(m * NF, NF)],
            sem).start()


def _wait_block_dmas(tv_hbm, tm_s, b, sem):
    for m in range(NMIC):
        pltpu.make_async_copy(
            tv_hbm.at[pl.ds(b * BE, BE), m, :, :],
            tm_s.at[pl.ds(b * BE, BE), :, pl.ds(m * NF, NF)],
            sem).wait()


def _gather_ratio(runidx_s, idx_smem, tm_s, sel_s, x2, sem):
    pltpu.make_async_copy(runidx_s, idx_smem, sem).start()
    pltpu.make_async_copy(runidx_s, idx_smem, sem).wait()

    def body(i, _):
        g = idx_smem[0, i]
        e = g // NAZI
        a = g % NAZI
        sel_s[pl.ds(i, 1), :] = tm_s[e, pl.ds(a, 1), :]
        return 0

    jax.lax.fori_loop(0, R, body, 0)
    sel = sel_s[...]
    num = jnp.sum(x2 * sel, axis=1, keepdims=True)
    den = jnp.sum(sel * sel, axis=1, keepdims=True)
    return num / den, sel


def _mega_kernel(x2_ref, doaT_ref, tv_hbm,
                 ss_ref, vad_ref, doa_ref,
                 tm_s, xT_s, x2T_s, sel_s, runmax_s, runidx_s,
                 doa1_s, ratio1_s, idx_smem, dma_sem, cp_sem):
    j = pl.program_id(0)

    @pl.when(j == 0)
    def _():
        _start_block_dmas(tv_hbm, tm_s, 0, dma_sem)
        _start_block_dmas(tv_hbm, tm_s, 1, dma_sem)
        xT_s[...] = x2_ref[...].T

    @pl.when(j < NBLK)
    def _():
        _wait_block_dmas(tv_hbm, tm_s, j, dma_sem)

        @pl.when(j < NBLK - 2)
        def _():
            _start_block_dmas(tv_hbm, tm_s, j + 2, dma_sem)

        for el in range(BE):
            e = j * BE + el
            m = _ele_dot(tm_s, e, xT_s)
            ss_ref[:, 0, el, :] = m.T
            _argmax_update(e, (j == 0) if el == 0 else False,
                           m, runmax_s, runidx_s)

    @pl.when(j == NBLK)
    def _():
        x2 = x2_ref[...]
        ratio, sel = _gather_ratio(runidx_s, idx_smem, tm_s, sel_s, x2,
                                   cp_sem)
        ratio1_s[...] = ratio
        doa1_s[...] = _doa_from_idx(runidx_s[...], doaT_ref)
        ipd2 = x2 - ratio * sel
        x2T_s[...] = ipd2
        xT_s[...] = ipd2.T

    @pl.when(j > NBLK)
    def _():
        jb = j - NBLK - 1
        for el in range(BE):
            e = jb * BE + el
            m = _ele_dot(tm_s, e, xT_s)
            _argmax_update(e, (jb == 0) if el == 0 else False,
                           m, runmax_s, runidx_s)

        @pl.when(j == 2 * NBLK)
        def _():
            ratio2, _ = _gather_ratio(runidx_s, idx_smem, tm_s, sel_s,
                                      x2T_s[...], cp_sem)
            vad_ref[...] = jnp.concatenate(
                [ratio1_s[...], ratio2], axis=1).reshape(4, 25, 2)
            doa2 = _doa_from_idx(runidx_s[...], doaT_ref)
            doa1 = doa1_s[...]
            dv = jnp.concatenate(
                [doa1[0:1, :].T, doa2[0:1, :].T,
                 doa1[1:2, :].T, doa2[1:2, :].T], axis=1)
            doa_ref[...] = dv.reshape(4, 25, 2, 2)


def kernel(pred_ipd, dpipd_template, doa_candidate):
    nb, nt, nf, nmic = pred_ipd.shape
    x2 = pred_ipd.transpose(0, 1, 3, 2).reshape(R, K)
    tv = dpipd_template.transpose(0, 3, 1, 2)
    doaT = doa_candidate.T

    ss, pred_VADs, pred_DOAs = pl.pallas_call(
        _mega_kernel,
        grid=(2 * NBLK + 1,),
        in_specs=[
            pl.BlockSpec((R, K), lambda j: (0, 0)),
            pl.BlockSpec((NAZI, 2), lambda j: (0, 0)),
            pl.BlockSpec(memory_space=pltpu.MemorySpace.HBM),
        ],
        out_specs=[
            pl.BlockSpec((R, 1, BE, NAZI),
                         lambda j: (0, jnp.minimum(j, NBLK - 1), 0, 0)),
            pl.BlockSpec((4, 25, 2), lambda j: (0, 0, 0)),
            pl.BlockSpec((4, 25, 2, 2), lambda j: (0, 0, 0, 0)),
        ],
        out_shape=[
            jax.ShapeDtypeStruct((R, NBLK, BE, NAZI), jnp.float32),
            jax.ShapeDtypeStruct((4, 25, 2), jnp.float32),
            jax.ShapeDtypeStruct((4, 25, 2, 2), jnp.float32),
        ],
        scratch_shapes=[
            pltpu.VMEM((NELE, NAZI, K), jnp.float32),
            pltpu.VMEM((K, R), jnp.float32),
            pltpu.VMEM((R, K), jnp.float32),
            pltpu.VMEM((R, K), jnp.float32),
            pltpu.VMEM((1, R), jnp.float32),
            pltpu.VMEM((1, R), jnp.int32),
            pltpu.VMEM((2, R), jnp.float32),
            pltpu.VMEM((R, 1), jnp.float32),
            pltpu.SMEM((1, R), jnp.int32),
            pltpu.SemaphoreType.DMA,
            pltpu.SemaphoreType.DMA,
        ],
        compiler_params=pltpu.CompilerParams(vmem_limit_bytes=61_000_000),
    )(x2, doaT, tv)

    pred_ss = ss.reshape(nb, nt, NELE, NAZI)
    return (pred_DOAs, pred_VADs, pred_ss)


# speedup vs baseline: 1.0599x; 1.0599x over previous
import jax
import jax.numpy as jnp
from jax.experimental import pallas as pl
from jax.experimental.pallas import tpu as pltpu

NELE = 90
NAZI = 90
G = NELE * NAZI
K = 256 * 6
R = 100
SCALE = 768.0
BE = 10
NBLK = NELE // BE
PA = 96
BROWS = BE * PA
NMIC = 6
NF = 256


def _doa_from_idx(ridx, doaT_ref):
    ele = ridx // NAZI
    azi = ridx % NAZI
    kk = jax.lax.broadcasted_iota(jnp.int32, (NAZI, 1), 0)
    elev = jnp.sum((kk == ele).astype(jnp.float32) * doaT_ref[:, 0:1],
                   axis=0, keepdims=True)
    aziv = jnp.sum((kk == azi).astype(jnp.float32) * doaT_ref[:, 1:2],
                   axis=0, keepdims=True)
    return jnp.concatenate([elev, aziv], axis=0)


def _ele_dot(tm_s, e, xT_s):
    return jax.lax.dot_general(
        tm_s[e], xT_s[...], (((1,), (0,)), ((), ())),
        preferred_element_type=jnp.float32) / SCALE


def _argmax_update(e, first, m, runmax_s, runidx_s):
    gidx = e * NAZI + jax.lax.broadcasted_iota(jnp.int32, (NAZI, 1), 0)
    bmax = jnp.max(m, axis=0, keepdims=True)
    bidx = jnp.min(jnp.where(m == bmax, gidx, jnp.int32(2**31 - 1)),
                   axis=0, keepdims=True)

    @pl.when(first)
    def _():
        runmax_s[...] = jnp.full((1, R), -jnp.inf, jnp.float32)
        runidx_s[...] = jnp.zeros((1, R), jnp.int32)

    better = bmax > runmax_s[...]
    runmax_s[...] = jnp.where(better, bmax, runmax_s[...])
    runidx_s[...] = jnp.where(better, bidx, runidx_s[...])


def _start_block_dmas(tv_hbm, tm_s, b, sem):
    for m in range(NMIC):
        pltpu.make_async_copy(
            tv_hbm.at[pl.ds(b * BE, BE), m, :, :],
            tm_s.at[pl.ds(b * BE, BE), :, pl.ds(m * NF, NF)],
            sem).start()


def _wait_block_dmas(tv_hbm, tm_s, b, sem):
    for m in range(NMIC):
        pltpu.make_async_copy(
            tv_hbm.at[pl.ds(b * BE, BE), m, :, :],
            tm_s.at[pl.ds(b * BE, BE), :, pl.ds(m * NF, NF)],
            sem).wait()


def _gather_ratio(runidx_s, idx_smem, tm_s, sel_s, x2, sem):
    pltpu.make_async_copy(runidx_s, idx_smem, sem).start()
    pltpu.make_async_copy(runidx_s, idx_smem, sem).wait()

    def body(i, _):
        g = idx_smem[0, i]
        e = g // NAZI
        a = g % NAZI
        sel_s[pl.ds(i, 1), :] = tm_s[e, pl.ds(a, 1), :]
        return 0

    jax.lax.fori_loop(0, R, body, 0)
    sel = sel_s[...]
    num = jnp.sum(x2 * sel, axis=1, keepdims=True)
    den = jnp.sum(sel * sel, axis=1, keepdims=True)
    return num / den, sel


def _mega_kernel(x2_ref, doaT_ref, tv_hbm,
                 ss_ref, vad_ref, doa_ref,
                 tm_s, xT_s, x2T_s, sel_s, runmax_s, runidx_s,
                 doa1_s, ratio1_s, idx_smem, dma_sem, cp_sem):
    j = pl.program_id(0)

    @pl.when(j == 0)
    def _():
        _start_block_dmas(tv_hbm, tm_s, 0, dma_sem)
        _start_block_dmas(tv_hbm, tm_s, 1, dma_sem)
        xT_s[...] = x2_ref[...].T

    @pl.when(j < NBLK)
    def _():
        _wait_block_dmas(tv_hbm, tm_s, j, dma_sem)

        @pl.when(j < NBLK - 2)
        def _():
            _start_block_dmas(tv_hbm, tm_s, j + 2, dma_sem)

        for el in range(BE):
            e = j * BE + el
            m = _ele_dot(tm_s, e, xT_s)
            ss_ref[:, 0, el, :] = m.T
            _argmax_update(e, (j == 0) if el == 0 else False,
                           m, runmax_s, runidx_s)

    @pl.when(j == NBLK)
    def _():
        x2 = x2_ref[...]
        ratio, sel = _gather_ratio(runidx_s, idx_smem, tm_s, sel_s, x2,
                                   cp_sem)
        ratio1_s[...] = ratio
        doa1_s[...] = _doa_from_idx(runidx_s[...], doaT_ref)
        ipd2 = x2 - ratio * sel
        x2T_s[...] = ipd2
        xT_s[...] = ipd2.T

    @pl.when(j > NBLK)
    def _():
        jb = j - NBLK - 1
        for el in range(BE):
            e = jb * BE + el
            m = _ele_dot(tm_s, e, xT_s)
            _argmax_update(e, (jb == 0) if el == 0 else False,
                           m, runmax_s, runidx_s)

        @pl.when(j == 2 * NBLK)
        def _():
            ratio2, _ = _gather_ratio(runidx_s, idx_smem, tm_s, sel_s,
                                      x2T_s[...], cp_sem)
            vad_ref[...] = jnp.concatenate(
                [ratio1_s[...], ratio2], axis=1).reshape(4, 25, 2)
            doa2 = _doa_from_idx(runidx_s[...], doaT_ref)
            doa1 = doa1_s[...]
            dv = jnp.concatenate(
                [doa1[0:1, :].T, doa2[0:1, :].T,
                 doa1[1:2, :].T, doa2[1:2, :].T], axis=1)
            doa_ref[...] = dv.reshape(4, 25, 2, 2)


def kernel(pred_ipd, dpipd_template, doa_candidate):
    nb, nt, nf, nmic = pred_ipd.shape
    x2 = pred_ipd.transpose(0, 1, 3, 2).reshape(R, K)
    tv = dpipd_template.transpose(0, 3, 1, 2)
    doaT = doa_candidate.T

    ss, pred_VADs, pred_DOAs = pl.pallas_call(
        _mega_kernel,
        grid=(2 * NBLK + 1,),
        in_specs=[
            pl.BlockSpec((R, K), lambda j: (0, 0)),
            pl.BlockSpec((NAZI, 2), lambda j: (0, 0)),
            pl.BlockSpec(memory_space=pltpu.MemorySpace.HBM),
        ],
        out_specs=[
            pl.BlockSpec((R, 1, BE, NAZI),
                         lambda j: (0, jnp.minimum(j, NBLK - 1), 0, 0)),
            pl.BlockSpec((4, 25, 2), lambda j: (0, 0, 0)),
            pl.BlockSpec((4, 25, 2, 2), lambda j: (0, 0, 0, 0)),
        ],
        out_shape=[
            jax.ShapeDtypeStruct((R, NBLK, BE, NAZI), jnp.float32),
            jax.ShapeDtypeStruct((4, 25, 2), jnp.float32),
            jax.ShapeDtypeStruct((4, 25, 2, 2), jnp.float32),
        ],
        scratch_shapes=[
            pltpu.VMEM((NELE, NAZI, K), jnp.float32),
            pltpu.VMEM((K, R), jnp.float32),
            pltpu.VMEM((R, K), jnp.float32),
            pltpu.VMEM((R, K), jnp.float32),
            pltpu.VMEM((1, R), jnp.float32),
            pltpu.VMEM((1, R), jnp.int32),
            pltpu.VMEM((2, R), jnp.float32),
            pltpu.VMEM((R, 1), jnp.float32),
            pltpu.SMEM((1, R), jnp.int32),
            pltpu.SemaphoreType.DMA,
            pltpu.SemaphoreType.DMA,
        ],
        compiler_params=pltpu.CompilerParams(vmem_limit_bytes=61_000_000),
    )(x2, doaT, tv)

    pred_ss = ss.reshape(nb, nt, NELE, NAZI)
    return (pred_DOAs, pred_VADs, pred_ss)


# speedup vs baseline: 1.0933x; 1.0315x over previous
import jax
import jax.numpy as jnp
from jax.experimental import pallas as pl
from jax.experimental.pallas import tpu as pltpu

NELE = 90
NAZI = 90
G = NELE * NAZI
K = 256 * 6
R = 100
SCALE = 768.0
BE = 18
NBLK = NELE // BE
PA = 96
BROWS = BE * PA
NMIC = 6
NF = 256


def _doa_from_idx(ridx, doaT_ref):
    ele = ridx // NAZI
    azi = ridx % NAZI
    kk = jax.lax.broadcasted_iota(jnp.int32, (NAZI, 1), 0)
    elev = jnp.sum((kk == ele).astype(jnp.float32) * doaT_ref[:, 0:1],
                   axis=0, keepdims=True)
    aziv = jnp.sum((kk == azi).astype(jnp.float32) * doaT_ref[:, 1:2],
                   axis=0, keepdims=True)
    return jnp.concatenate([elev, aziv], axis=0)


def _ele_dot(tm_s, e, xT_s):
    return jax.lax.dot_general(
        tm_s[e], xT_s[...], (((1,), (0,)), ((), ())),
        preferred_element_type=jnp.float32) / SCALE


def _argmax_update(e, first, m, runmax_s, runidx_s):
    gidx = e * NAZI + jax.lax.broadcasted_iota(jnp.int32, (NAZI, 1), 0)
    bmax = jnp.max(m, axis=0, keepdims=True)
    bidx = jnp.min(jnp.where(m == bmax, gidx, jnp.int32(2**31 - 1)),
                   axis=0, keepdims=True)

    @pl.when(first)
    def _():
        runmax_s[...] = jnp.full((1, R), -jnp.inf, jnp.float32)
        runidx_s[...] = jnp.zeros((1, R), jnp.int32)

    better = bmax > runmax_s[...]
    runmax_s[...] = jnp.where(better, bmax, runmax_s[...])
    runidx_s[...] = jnp.where(better, bidx, runidx_s[...])


def _start_block_dmas(tv_hbm, tm_s, b, sem):
    for m in range(NMIC):
        pltpu.make_async_copy(
            tv_hbm.at[pl.ds(b * BE, BE), m, :, :],
            tm_s.at[pl.ds(b * BE, BE), :, pl.ds(m * NF, NF)],
            sem).start()


def _wait_block_dmas(tv_hbm, tm_s, b, sem):
    for m in range(NMIC):
        pltpu.make_async_copy(
            tv_hbm.at[pl.ds(b * BE, BE), m, :, :],
            tm_s.at[pl.ds(b * BE, BE), :, pl.ds(m * NF, NF)],
            sem).wait()


def _gather_ratio(runidx_s, idx_smem, tm_s, sel_s, x2, sem):
    pltpu.make_async_copy(runidx_s, idx_smem, sem).start()
    pltpu.make_async_copy(runidx_s, idx_smem, sem).wait()

    def body(i, _):
        g = idx_smem[0, i]
        e = g // NAZI
        a = g % NAZI
        sel_s[pl.ds(i, 1), :] = tm_s[e, pl.ds(a, 1), :]
        return 0

    jax.lax.fori_loop(0, R, body, 0)
    sel = sel_s[...]
    num = jnp.sum(x2 * sel, axis=1, keepdims=True)
    den = jnp.sum(sel * sel, axis=1, keepdims=True)
    return num / den, sel


def _mega_kernel(x2_ref, doaT_ref, tv_hbm,
                 ss_ref, vad_ref, doa_ref,
                 tm_s, xT_s, x2T_s, sel_s, runmax_s, runidx_s,
                 doa1_s, ratio1_s, idx_smem, dma_sem, cp_sem):
    j = pl.program_id(0)

    @pl.when(j == 0)
    def _():
        _start_block_dmas(tv_hbm, tm_s, 0, dma_sem)
        _start_block_dmas(tv_hbm, tm_s, 1, dma_sem)
        xT_s[...] = x2_ref[...].T

    @pl.when(j < NBLK)
    def _():
        _wait_block_dmas(tv_hbm, tm_s, j, dma_sem)

        @pl.when(j < NBLK - 2)
        def _():
            _start_block_dmas(tv_hbm, tm_s, j + 2, dma_sem)

        for el in range(BE):
            e = j * BE + el
            m = _ele_dot(tm_s, e, xT_s)
            ss_ref[:, 0, el, :] = m.T
            _argmax_update(e, (j == 0) if el == 0 else False,
                           m, runmax_s, runidx_s)

    @pl.when(j == NBLK)
    def _():
        x2 = x2_ref[...]
        ratio, sel = _gather_ratio(runidx_s, idx_smem, tm_s, sel_s, x2,
                                   cp_sem)
        ratio1_s[...] = ratio
        doa1_s[...] = _doa_from_idx(runidx_s[...], doaT_ref)
        ipd2 = x2 - ratio * sel
        x2T_s[...] = ipd2
        xT_s[...] = ipd2.T

    @pl.when(j > NBLK)
    def _():
        jb = j - NBLK - 1
        for el in range(BE):
            e = jb * BE + el
            m = _ele_dot(tm_s, e, xT_s)
            _argmax_update(e, (jb == 0) if el == 0 else False,
                           m, runmax_s, runidx_s)

        @pl.when(j == 2 * NBLK)
        def _():
            ratio2, _ = _gather_ratio(runidx_s, idx_smem, tm_s, sel_s,
                                      x2T_s[...], cp_sem)
            vad_ref[...] = jnp.concatenate(
                [ratio1_s[...], ratio2], axis=1).reshape(4, 25, 2)
            doa2 = _doa_from_idx(runidx_s[...], doaT_ref)
            doa1 = doa1_s[...]
            dv = jnp.concatenate(
                [doa1[0:1, :].T, doa2[0:1, :].T,
                 doa1[1:2, :].T, doa2[1:2, :].T], axis=1)
            doa_ref[...] = dv.reshape(4, 25, 2, 2)


def kernel(pred_ipd, dpipd_template, doa_candidate):
    nb, nt, nf, nmic = pred_ipd.shape
    x2 = pred_ipd.transpose(0, 1, 3, 2).reshape(R, K)
    tv = dpipd_template.transpose(0, 3, 1, 2)
    doaT = doa_candidate.T

    ss, pred_VADs, pred_DOAs = pl.pallas_call(
        _mega_kernel,
        grid=(2 * NBLK + 1,),
        in_specs=[
            pl.BlockSpec((R, K), lambda j: (0, 0)),
            pl.BlockSpec((NAZI, 2), lambda j: (0, 0)),
            pl.BlockSpec(memory_space=pltpu.MemorySpace.HBM),
        ],
        out_specs=[
            pl.BlockSpec((R, 1, BE, NAZI),
                         lambda j: (0, jnp.minimum(j, NBLK - 1), 0, 0)),
            pl.BlockSpec((4, 25, 2), lambda j: (0, 0, 0)),
            pl.BlockSpec((4, 25, 2, 2), lambda j: (0, 0, 0, 0)),
        ],
        out_shape=[
            jax.ShapeDtypeStruct((R, NBLK, BE, NAZI), jnp.float32),
            jax.ShapeDtypeStruct((4, 25, 2), jnp.float32),
            jax.ShapeDtypeStruct((4, 25, 2, 2), jnp.float32),
        ],
        scratch_shapes=[
            pltpu.VMEM((NELE, NAZI, K), jnp.float32),
            pltpu.VMEM((K, R), jnp.float32),
            pltpu.VMEM((R, K), jnp.float32),
            pltpu.VMEM((R, K), jnp.float32),
            pltpu.VMEM((1, R), jnp.float32),
            pltpu.VMEM((1, R), jnp.int32),
            pltpu.VMEM((2, R), jnp.float32),
            pltpu.VMEM((R, 1), jnp.float32),
            pltpu.SMEM((1, R), jnp.int32),
            pltpu.SemaphoreType.DMA,
            pltpu.SemaphoreType.DMA,
        ],
        compiler_params=pltpu.CompilerParams(vmem_limit_bytes=61_000_000),
    )(x2, doaT, tv)

    pred_ss = ss.reshape(nb, nt, NELE, NAZI)
    return (pred_DOAs, pred_VADs, pred_ss)


# speedup vs baseline: 1.1009x; 1.0070x over previous
import jax
import jax.numpy as jnp
from jax.experimental import pallas as pl
from jax.experimental.pallas import tpu as pltpu

NELE = 90
NAZI = 90
G = NELE * NAZI
K = 256 * 6
R = 100
SCALE = 768.0
BE = 15
NBLK = NELE // BE
PA = 96
BROWS = BE * PA
NMIC = 6
NF = 256


def _doa_from_idx(ridx, doaT_ref):
    ele = ridx // NAZI
    azi = ridx % NAZI
    kk = jax.lax.broadcasted_iota(jnp.int32, (NAZI, 1), 0)
    elev = jnp.sum((kk == ele).astype(jnp.float32) * doaT_ref[:, 0:1],
                   axis=0, keepdims=True)
    aziv = jnp.sum((kk == azi).astype(jnp.float32) * doaT_ref[:, 1:2],
                   axis=0, keepdims=True)
    return jnp.concatenate([elev, aziv], axis=0)


def _ele_dot(tm_s, e, xT_s):
    return jax.lax.dot_general(
        tm_s[e], xT_s[...], (((1,), (0,)), ((), ())),
        preferred_element_type=jnp.float32) / SCALE


def _argmax_update(e, first, m, runmax_s, runidx_s):
    gidx = e * NAZI + jax.lax.broadcasted_iota(jnp.int32, (NAZI, 1), 0)
    bmax = jnp.max(m, axis=0, keepdims=True)
    bidx = jnp.min(jnp.where(m == bmax, gidx, jnp.int32(2**31 - 1)),
                   axis=0, keepdims=True)

    @pl.when(first)
    def _():
        runmax_s[...] = jnp.full((1, R), -jnp.inf, jnp.float32)
        runidx_s[...] = jnp.zeros((1, R), jnp.int32)

    better = bmax > runmax_s[...]
    runmax_s[...] = jnp.where(better, bmax, runmax_s[...])
    runidx_s[...] = jnp.where(better, bidx, runidx_s[...])


def _start_block_dmas(tv_hbm, tm_s, b, sem):
    for m in range(NMIC):
        pltpu.make_async_copy(
            tv_hbm.at[pl.ds(b * BE, BE), m, :, :],
            tm_s.at[pl.ds(b * BE, BE), :, pl.ds(m * NF, NF)],
            sem).start()


def _wait_block_dmas(tv_hbm, tm_s, b, sem):
    for m in range(NMIC):
        pltpu.make_async_copy(
            tv_hbm.at[pl.ds(b * BE, BE), m, :, :],
            tm_s.at[pl.ds(b * BE, BE), :, pl.ds(m * NF, NF)],
            sem).wait()


def _gather_ratio(runidx_s, idx_smem, tm_s, sel_s, x2, sem):
    pltpu.make_async_copy(runidx_s, idx_smem, sem).start()
    pltpu.make_async_copy(runidx_s, idx_smem, sem).wait()

    def body(i, _):
        g = idx_smem[0, i]
        e = g // NAZI
        a = g % NAZI
        sel_s[pl.ds(i, 1), :] = tm_s[e, pl.ds(a, 1), :]
        return 0

    jax.lax.fori_loop(0, R, body, 0)
    sel = sel_s[...]
    num = jnp.sum(x2 * sel, axis=1, keepdims=True)
    den = jnp.sum(sel * sel, axis=1, keepdims=True)
    return num / den, sel


def _mega_kernel(x2_ref, doaT_ref, tv_hbm,
                 ss_ref, vad_ref, doa_ref,
                 tm_s, xT_s, x2T_s, sel_s, runmax_s, runidx_s,
                 doa1_s, ratio1_s, idx_smem, dma_sem, cp_sem):
    j = pl.program_id(0)

    @pl.when(j == 0)
    def _():
        _start_block_dmas(tv_hbm, tm_s, 0, dma_sem)
        _start_block_dmas(tv_hbm, tm_s, 1, dma_sem)
        xT_s[...] = x2_ref[...].T

    @pl.when(j < NBLK)
    def _():
        _wait_block_dmas(tv_hbm, tm_s, j, dma_sem)

        @pl.when(j < NBLK - 2)
        def _():
            _start_block_dmas(tv_hbm, tm_s, j + 2, dma_sem)

        for el in range(BE):
            e = j * BE + el
            m = _ele_dot(tm_s, e, xT_s)
            ss_ref[:, 0, el, :] = m.T
            _argmax_update(e, (j == 0) if el == 0 else False,
                           m, runmax_s, runidx_s)

    @pl.when(j == NBLK)
    def _():
        x2 = x2_ref[...]
        ratio, sel = _gather_ratio(runidx_s, idx_smem, tm_s, sel_s, x2,
                                   cp_sem)
        ratio1_s[...] = ratio
        doa1_s[...] = _doa_from_idx(runidx_s[...], doaT_ref)
        ipd2 = x2 - ratio * sel
        x2T_s[...] = ipd2
        xT_s[...] = ipd2.T

    @pl.when(j > NBLK)
    def _():
        jb = j - NBLK - 1
        for el in range(BE):
            e = jb * BE + el
            m = _ele_dot(tm_s, e, xT_s)
            _argmax_update(e, (jb == 0) if el == 0 else False,
                           m, runmax_s, runidx_s)

        @pl.when(j == 2 * NBLK)
        def _():
            ratio2, _ = _gather_ratio(runidx_s, idx_smem, tm_s, sel_s,
                                      x2T_s[...], cp_sem)
            vad_ref[...] = jnp.concatenate(
                [ratio1_s[...], ratio2], axis=1).reshape(4, 25, 2)
            doa2 = _doa_from_idx(runidx_s[...], doaT_ref)
            doa1 = doa1_s[...]
            dv = jnp.concatenate(
                [doa1[0:1, :].T, doa2[0:1, :].T,
                 doa1[1:2, :].T, doa2[1:2, :].T], axis=1)
            doa_ref[...] = dv.reshape(4, 25, 2, 2)


def kernel(pred_ipd, dpipd_template, doa_candidate):
    nb, nt, nf, nmic = pred_ipd.shape
    x2 = pred_ipd.transpose(0, 1, 3, 2).reshape(R, K)
    tv = dpipd_template.transpose(0, 3, 1, 2)
    doaT = doa_candidate.T

    ss, pred_VADs, pred_DOAs = pl.pallas_call(
        _mega_kernel,
        grid=(2 * NBLK + 1,),
        in_specs=[
            pl.BlockSpec((R, K), lambda j: (0, 0)),
            pl.BlockSpec((NAZI, 2), lambda j: (0, 0)),
            pl.BlockSpec(memory_space=pltpu.MemorySpace.HBM),
        ],
        out_specs=[
            pl.BlockSpec((R, 1, BE, NAZI),
                         lambda j: (0, jnp.minimum(j, NBLK - 1), 0, 0)),
            pl.BlockSpec((4, 25, 2), lambda j: (0, 0, 0)),
            pl.BlockSpec((4, 25, 2, 2), lambda j: (0, 0, 0, 0)),
        ],
        out_shape=[
            jax.ShapeDtypeStruct((R, NBLK, BE, NAZI), jnp.float32),
            jax.ShapeDtypeStruct((4, 25, 2), jnp.float32),
            jax.ShapeDtypeStruct((4, 25, 2, 2), jnp.float32),
        ],
        scratch_shapes=[
            pltpu.VMEM((NELE, NAZI, K), jnp.float32),
            pltpu.VMEM((K, R), jnp.float32),
            pltpu.VMEM((R, K), jnp.float32),
            pltpu.VMEM((R, K), jnp.float32),
            pltpu.VMEM((1, R), jnp.float32),
            pltpu.VMEM((1, R), jnp.int32),
            pltpu.VMEM((2, R), jnp.float32),
            pltpu.VMEM((R, 1), jnp.float32),
            pltpu.SMEM((1, R), jnp.int32),
            pltpu.SemaphoreType.DMA,
            pltpu.SemaphoreType.DMA,
        ],
        compiler_params=pltpu.CompilerParams(vmem_limit_bytes=61_000_000),
    )(x2, doaT, tv)

    pred_ss = ss.reshape(nb, nt, NELE, NAZI)
    return (pred_DOAs, pred_VADs, pred_ss)
